# Initial kernel scaffold; baseline (speedup 1.0000x reference)
import jax
import jax.numpy as jnp
from jax.experimental import pallas as pl


def kernel(inp):
    raise NotImplementedError("write your pallas kernel here")


# speedup vs baseline: 7.5557x; 7.5557x over previous
import functools

import jax
import jax.numpy as jnp
from jax import lax
from jax.experimental import pallas as pl
from jax.experimental.pallas import tpu as pltpu
from jax.experimental.pallas import tpu_sc as plsc

H = 512
W = 512
C = 96
SH = 36
SW = 36
L = 16
NCV = C // L
ROW = W * C
GROW = SW * C

_mesh = plsc.VectorSubcoreMesh(core_axis_name="c", subcore_axis_name="s",
                               num_cores=2, num_subcores=16)


def _zero_buf(ref, nwords):
    z = jnp.zeros((L,), jnp.float32)

    def body(i, _):
        ref[pl.ds(i * L, L)] = z
        return 0

    lax.fori_loop(0, nwords // L, body, 0)


def _pool_row_into_slab(buf, slab):
    def edge(x0, nx, col):
        for cv in range(NCV):
            a = buf[pl.ds(x0 * C + cv * L, L)]
            for j in range(1, nx):
                a = a + buf[pl.ds((x0 + j) * C + cv * L, L)]
            plsc.addupdate(slab.at[pl.ds(col * C + cv * L, L)], a)

    edge(0, 8, 2)
    edge(504, 8, 34)

    def body(xb, _):
        bx = (xb * 16 - 8) * C
        col = (xb + 2) * C
        for cv in range(NCV):
            a = buf[pl.ds(bx + cv * L, L)]
            for j in range(1, 16):
                a = a + buf[pl.ds(bx + j * C + cv * L, L)]
            plsc.addupdate(slab.at[pl.ds(col + cv * L, L)], a)
        return 0

    lax.fori_loop(1, 32, body, 0)


def _conv5_row(slab, fs):
    w5 = (0.0625, 0.25, 0.375, 0.25, 0.0625)

    def body(k, _):
        base = (k - 2) * C
        for cv in range(NCV):
            a = slab[pl.ds(base + cv * L, L)] * w5[0]
            for d in range(1, 5):
                a = a + slab[pl.ds(base + d * C + cv * L, L)] * w5[d]
            fs[pl.ds(k * C + cv * L, L)] = a
        return 0

    lax.fori_loop(2, 34, body, 0)
    for cv in range(NCV):
        a = (slab[pl.ds(32 * C + cv * L, L)] * 0.0625
             + slab[pl.ds(33 * C + cv * L, L)] * 0.25
             + slab[pl.ds(34 * C + cv * L, L)] * 0.3125)
        fs[pl.ds(34 * C + cv * L, L)] = a


def _splat_body(inp_hbm, f_hbm, buf, slab, fs):
    wid = lax.axis_index("c") * 16 + lax.axis_index("s")

    def do_bin(y0, nrows, b):
        _zero_buf(slab, GROW)

        def row_body(ry, _):
            off = pl.multiple_of((y0 + ry) * ROW, 8)
            pltpu.sync_copy(inp_hbm.at[pl.ds(off, ROW)], buf)
            _pool_row_into_slab(buf, slab)
            return 0

        lax.fori_loop(0, nrows, row_body, 0)
        _conv5_row(slab, fs)
        pltpu.sync_copy(fs, f_hbm.at[pl.ds(b * GROW, GROW)])

    _zero_buf(fs, GROW)

    @pl.when(wid == 0)
    def _():
        do_bin(0, 8, 2)
        do_bin(504, 8, 34)

    @pl.when(wid > 0)
    def _():
        do_bin(16 * wid - 8, 16, wid + 2)


_splat = pl.kernel(
    _splat_body,
    out_type=jax.ShapeDtypeStruct((SH * GROW,), jnp.float32),
    mesh=_mesh,
    scratch_types=[
        pltpu.VMEM((ROW,), jnp.float32),
        pltpu.VMEM((GROW,), jnp.float32),
        pltpu.VMEM((GROW,), jnp.float32),
    ],
)


def _slice_body(f_hbm, out_hbm, fbuf, rbuf, obuf):
    wid = lax.axis_index("c") * 16 + lax.axis_index("s")
    foff = pl.multiple_of((wid + 2) * GROW, 8)
    pltpu.sync_copy(f_hbm.at[pl.ds(foff, 2 * GROW)], fbuf)

    def s_body(s, _):
        ays = s.astype(jnp.float32) * 0.0625

        def r_body(i, _):
            v0 = fbuf[pl.ds(i * L, L)]
            v1 = fbuf[pl.ds(GROW + i * L, L)]
            rbuf[pl.ds(i * L, L)] = v0 + (v1 - v0) * ays
            return 0

        lax.fori_loop(0, GROW // L, r_body, 0)

        def q_body(q, _):
            abase = (q + 2) * C
            obase = q * 16 * C
            for cv in range(NCV):
                a = rbuf[pl.ds(abase + cv * L, L)]
                b = rbuf[pl.ds(abase + C + cv * L, L)]
                d = b - a
                for r in range(16):
                    obuf[pl.ds(obase + r * C + cv * L, L)] = a + d * (r * 0.0625)
            return 0

        lax.fori_loop(0, 32, q_body, 0)
        ooff = pl.multiple_of((16 * wid + s) * ROW, 8)
        pltpu.sync_copy(obuf, out_hbm.at[pl.ds(ooff, ROW)])
        return 0

    lax.fori_loop(0, 16, s_body, 0)


_slice = pl.kernel(
    _slice_body,
    out_type=jax.ShapeDtypeStruct((H * W * C,), jnp.float32),
    mesh=_mesh,
    scratch_types=[
        pltpu.VMEM((2 * GROW,), jnp.float32),
        pltpu.VMEM((GROW,), jnp.float32),
        pltpu.VMEM((ROW,), jnp.float32),
    ],
)


def kernel(inp):
    inp_flat = inp.reshape(H * W * C)
    f = _splat(inp_flat)
    out_flat = _slice(f)
    return out_flat.reshape(H, W, C)


# speedup vs baseline: 10.9387x; 1.4477x over previous
import functools

import jax
import jax.numpy as jnp
from jax import lax
from jax.experimental import pallas as pl
from jax.experimental.pallas import tpu as pltpu
from jax.experimental.pallas import tpu_sc as plsc

H = 512
W = 512
C = 96
SH = 36
SW = 36
L = 16
NCV = C // L
ROW = W * C
GROW = SW * C

_mesh = plsc.VectorSubcoreMesh(core_axis_name="c", subcore_axis_name="s",
                               num_cores=2, num_subcores=16)


def _zero_buf(ref, nwords):
    z = jnp.zeros((L,), jnp.float32)

    def body(i, _):
        ref[pl.ds(i * L, L)] = z
        return 0

    lax.fori_loop(0, nwords // L, body, 0)


def _pool_row_into_slab(buf, slab):
    def edge(x0, nx, col):
        for cv in range(NCV):
            a = buf[0, x0, pl.ds(cv * L, L)]
            for j in range(1, nx):
                a = a + buf[0, x0 + j, pl.ds(cv * L, L)]
            plsc.addupdate(slab.at[pl.ds(col * C + cv * L, L)], a)

    edge(0, 8, 2)
    edge(504, 8, 34)

    def body(xb, _):
        x0 = xb * 16 - 8
        col = (xb + 2) * C
        for cv in range(NCV):
            a = buf[0, x0, pl.ds(cv * L, L)]
            for j in range(1, 16):
                a = a + buf[0, x0 + j, pl.ds(cv * L, L)]
            plsc.addupdate(slab.at[pl.ds(col + cv * L, L)], a)
        return 0

    lax.fori_loop(1, 32, body, 0)


def _conv5_row(slab, fs):
    w5 = (0.0625, 0.25, 0.375, 0.25, 0.0625)

    def body(k, _):
        base = (k - 2) * C
        for cv in range(NCV):
            a = slab[pl.ds(base + cv * L, L)] * w5[0]
            for d in range(1, 5):
                a = a + slab[pl.ds(base + d * C + cv * L, L)] * w5[d]
            fs[pl.ds(k * C + cv * L, L)] = a
        return 0

    lax.fori_loop(2, 34, body, 0)
    for cv in range(NCV):
        a = (slab[pl.ds(32 * C + cv * L, L)] * 0.0625
             + slab[pl.ds(33 * C + cv * L, L)] * 0.25
             + slab[pl.ds(34 * C + cv * L, L)] * 0.3125)
        fs[pl.ds(34 * C + cv * L, L)] = a


def _splat_body(inp_hbm, f_hbm, buf, slab, fs):
    wid = lax.axis_index("c") * 16 + lax.axis_index("s")

    def do_bin(y0, nrows, b):
        _zero_buf(slab, GROW)

        def row_body(ry, _):
            pltpu.sync_copy(inp_hbm.at[pl.ds(y0 + ry, 1)], buf)
            _pool_row_into_slab(buf, slab)
            return 0

        lax.fori_loop(0, nrows, row_body, 0)
        _conv5_row(slab, fs)
        pltpu.sync_copy(fs, f_hbm.at[pl.ds(b * GROW, GROW)])

    _zero_buf(fs, GROW)

    @pl.when(wid == 0)
    def _():
        do_bin(0, 8, 2)
        do_bin(504, 8, 34)

    @pl.when(wid > 0)
    def _():
        do_bin(16 * wid - 8, 16, wid + 2)


_splat = pl.kernel(
    _splat_body,
    out_type=jax.ShapeDtypeStruct((SH * GROW,), jnp.float32),
    mesh=_mesh,
    scratch_types=[
        pltpu.VMEM((1, W, C), jnp.float32),
        pltpu.VMEM((GROW,), jnp.float32),
        pltpu.VMEM((GROW,), jnp.float32),
    ],
)


def _slice_body(f_hbm, out_hbm, fbuf, rbuf, obuf):
    wid = lax.axis_index("c") * 16 + lax.axis_index("s")
    foff = pl.multiple_of((wid + 2) * GROW, 8)
    pltpu.sync_copy(f_hbm.at[pl.ds(foff, 2 * GROW)], fbuf)

    def s_body(s, _):
        ays = s.astype(jnp.float32) * 0.0625

        def r_body(i, _):
            v0 = fbuf[pl.ds(i * L, L)]
            v1 = fbuf[pl.ds(GROW + i * L, L)]
            rbuf[pl.ds(i * L, L)] = v0 + (v1 - v0) * ays
            return 0

        lax.fori_loop(0, GROW // L, r_body, 0)

        def q_body(q, _):
            abase = (q + 2) * C
            x0 = q * 16
            for cv in range(NCV):
                a = rbuf[pl.ds(abase + cv * L, L)]
                b = rbuf[pl.ds(abase + C + cv * L, L)]
                d = b - a
                for r in range(16):
                    obuf[0, x0 + r, pl.ds(cv * L, L)] = a + d * (r * 0.0625)
            return 0

        lax.fori_loop(0, 32, q_body, 0)
        pltpu.sync_copy(obuf, out_hbm.at[pl.ds(16 * wid + s, 1)])
        return 0

    lax.fori_loop(0, 16, s_body, 0)


_slice = pl.kernel(
    _slice_body,
    out_type=jax.ShapeDtypeStruct((H, W, C), jnp.float32),
    mesh=_mesh,
    scratch_types=[
        pltpu.VMEM((2 * GROW,), jnp.float32),
        pltpu.VMEM((GROW,), jnp.float32),
        pltpu.VMEM((1, W, C), jnp.float32),
    ],
)


def kernel(inp):
    f = _splat(inp)
    return _slice(f)
